# baseline (speedup 1.0000x reference)
import functools

import jax
import jax.numpy as jnp
from jax import lax
from jax.experimental import pallas as pl
from jax.experimental.pallas import tpu as pltpu
from jax.experimental.pallas import tpu_sc as plsc

_B = 16384
_NEG = 20
_D = 64
_V = 1000000
_NW = 32
_BPW = _B // _NW
_CB = 16
_NCH = _BPW // _CB
_NEGC = _CB * _NEG


def _sc_scores(tid1, cid1, nid1, ttab, ctab):
    mesh = plsc.VectorSubcoreMesh(core_axis_name="c", subcore_axis_name="s")

    @functools.partial(
        pl.kernel,
        mesh=mesh,
        compiler_params=pltpu.CompilerParams(
            needs_layout_passes=False, use_tc_tiling_on_sc=True
        ),
        out_type=[
            jax.ShapeDtypeStruct((_B,), jnp.float32),
            jax.ShapeDtypeStruct((_B * _NEG,), jnp.float32),
        ],
        scratch_types=[
            pltpu.VMEM((_BPW + 16,), jnp.int32),
            pltpu.VMEM((_BPW + 16,), jnp.int32),
            pltpu.VMEM((_BPW * _NEG + 16,), jnp.int32),
            pltpu.VMEM((_CB, _D), jnp.float32),
            pltpu.VMEM((_CB, _D), jnp.float32),
            pltpu.VMEM((_CB, _D), jnp.float32),
            pltpu.VMEM((_CB, _D), jnp.float32),
            pltpu.VMEM((_NEGC, _D), jnp.float32),
            pltpu.VMEM((_NEGC, _D), jnp.float32),
            pltpu.VMEM((_CB,), jnp.float32),
            pltpu.VMEM((_NEGC,), jnp.float32),
            pltpu.SemaphoreType.DMA,
            pltpu.SemaphoreType.DMA,
        ],
    )
    def k(tid_h, cid_h, nid_h, ttab_h, ctab_h, pos_h, neg_h,
          tidx, cidx, nidx, trow0, trow1, crow0, crow1, nrow0, nrow1,
          posb, negb, sem0, sem1):
        wid = lax.axis_index("s") * 2 + lax.axis_index("c")
        pltpu.sync_copy(tid_h.at[pl.ds(wid * _BPW, _BPW)],
                        tidx.at[pl.ds(0, _BPW)])
        pltpu.sync_copy(cid_h.at[pl.ds(wid * _BPW, _BPW)],
                        cidx.at[pl.ds(0, _BPW)])
        pltpu.sync_copy(nid_h.at[pl.ds(wid * _BPW * _NEG, _BPW * _NEG)],
                        nidx.at[pl.ds(0, _BPW * _NEG)])

        lanes = lax.iota(jnp.int32, 16)
        bufs = ((trow0, crow0, nrow0, sem0), (trow1, crow1, nrow1, sem1))

        def fire(c, buf):
            trow, crow, nrow, sem = buf

            def fire_item(i, carry2):
                traw = tidx[pl.ds(c * _CB + i, 16)][0]
                craw = cidx[pl.ds(c * _CB + i, 16)][0]
                pltpu.async_copy(ttab_h.at[traw], trow.at[i], sem)
                pltpu.async_copy(
                    ctab_h.at[lax.shift_right_logical(craw, 3), craw & 7],
                    crow.at[i], sem)
                for n in range(_NEG):
                    nraw = nidx[pl.ds(c * _NEGC + i * _NEG + n, 16)][0]
                    pltpu.async_copy(
                        ctab_h.at[lax.shift_right_logical(nraw, 3), nraw & 7],
                        nrow.at[i * _NEG + n], sem)
                return carry2

            lax.fori_loop(0, _CB, fire_item, 0)

        def drain(buf):
            trow, crow, nrow, sem = buf

            def drain_item(i, carry2):
                pltpu.make_async_copy(ttab_h.at[0], trow.at[i],
                                      sem).wait()
                pltpu.make_async_copy(ctab_h.at[0, 0], crow.at[i],
                                      sem).wait()
                for n in range(_NEG):
                    pltpu.make_async_copy(ctab_h.at[0, 0],
                                          nrow.at[i * _NEG + n], sem).wait()
                return carry2

            lax.fori_loop(0, _CB, drain_item, 0)

        def compute(c, buf):
            trow, crow, nrow, _ = buf

            def lane_sum(vecs):
                return jnp.sum(vecs[0] + vecs[1] + vecs[2] + vecs[3])

            for g in range(_CB // 16):
                def item(i0, accs):
                    i = i0 + g * 16
                    onlane = lanes == i0
                    t = [trow[i, pl.ds(kk * 16, 16)] for kk in range(4)]
                    cv = [crow[i, pl.ds(kk * 16, 16)] for kk in range(4)]
                    new = [jnp.where(
                        onlane, lane_sum([t[kk] * cv[kk] for kk in range(4)]),
                        accs[0])]
                    for n in range(_NEG):
                        nv = [nrow[i * _NEG + n, pl.ds(kk * 16, 16)]
                              for kk in range(4)]
                        new.append(jnp.where(
                            onlane,
                            lane_sum([t[kk] * nv[kk] for kk in range(4)]),
                            accs[n + 1]))
                    return tuple(new)

                zeros = jnp.zeros((16,), jnp.float32)
                accs = lax.fori_loop(0, 16, item, (zeros,) * (_NEG + 1))
                posb[pl.ds(g * 16, 16)] = accs[0]
                for n in range(_NEG):
                    negb[pl.ds(n * _CB + g * 16, 16)] = accs[n + 1]

            base = wid * _BPW + c * _CB
            pltpu.sync_copy(posb, pos_h.at[pl.ds(base, _CB)])
            pltpu.sync_copy(negb, neg_h.at[pl.ds(base * _NEG, _NEGC)])

        fire(0, bufs[0])

        def pair(p, carry):
            for b in range(2):
                cur = p * 2 + b

                @pl.when(cur + 1 < _NCH)
                def _():
                    fire(cur + 1, bufs[1 - b])

                drain(bufs[b])
                compute(cur, bufs[b])
            return carry

        lax.fori_loop(0, _NCH // 2, pair, 0)

    return k(tid1, cid1, nid1, ttab, ctab)


def _tc_loss(pos2d, neg2d):
    def body(p_ref, n_ref, o_ref):
        p = p_ref[...]
        n = n_ref[...]

        def logsig(x):
            return jnp.minimum(x, 0.0) - jnp.log1p(jnp.exp(-jnp.abs(x)))

        total = jnp.sum(logsig(p)) + jnp.sum(logsig(-n))
        o_ref[0, 0] = -total / _B

    out = pl.pallas_call(
        body,
        out_shape=jax.ShapeDtypeStruct((1, 1), jnp.float32),
        out_specs=pl.BlockSpec(memory_space=pltpu.SMEM),
    )(pos2d, neg2d)
    return out[0, 0]


def kernel(target_ids, context_ids, negative_ids, target_embed, context_embed):
    tid1 = target_ids.astype(jnp.int32)
    cid1 = context_ids.astype(jnp.int32)
    nid1 = negative_ids.astype(jnp.int32).reshape(-1)
    pos, negf = _sc_scores(tid1, cid1, nid1, target_embed,
                           context_embed.reshape(_V // 8, 8, _D))
    return _tc_loss(pos.reshape(128, 128), negf.reshape(1280, 256))


# speedup vs baseline: 1.2680x; 1.2680x over previous
import functools

import jax
import jax.numpy as jnp
from jax import lax
from jax.experimental import pallas as pl
from jax.experimental.pallas import tpu as pltpu
from jax.experimental.pallas import tpu_sc as plsc

_B = 16384
_NEG = 20
_D = 64
_V = 1000000
_NW = 32
_BPW = _B // _NW
_CB = 16
_NCH = _BPW // _CB
_NEGC = _CB * _NEG


def _sc_scores(tid1, cid1, nid1, ttab, ctab, dum):
    mesh = plsc.VectorSubcoreMesh(core_axis_name="c", subcore_axis_name="s")

    @functools.partial(
        pl.kernel,
        mesh=mesh,
        compiler_params=pltpu.CompilerParams(
            needs_layout_passes=False, use_tc_tiling_on_sc=True
        ),
        out_type=[
            jax.ShapeDtypeStruct((_B,), jnp.float32),
            jax.ShapeDtypeStruct((_B * _NEG,), jnp.float32),
        ],
        scratch_types=[
            pltpu.VMEM((_BPW + 16,), jnp.int32),
            pltpu.VMEM((_BPW + 16,), jnp.int32),
            pltpu.VMEM((_BPW * _NEG + 16,), jnp.int32),
            pltpu.VMEM((_CB, _D), jnp.float32),
            pltpu.VMEM((_CB, _D), jnp.float32),
            pltpu.VMEM((_CB, _D), jnp.float32),
            pltpu.VMEM((_CB, _D), jnp.float32),
            pltpu.VMEM((_NEGC, _D), jnp.float32),
            pltpu.VMEM((_NEGC, _D), jnp.float32),
            pltpu.VMEM((_CB,), jnp.float32),
            pltpu.VMEM((_NEGC,), jnp.float32),
            pltpu.SemaphoreType.DMA,
            pltpu.SemaphoreType.DMA,
        ],
    )
    def k(tid_h, cid_h, nid_h, ttab_h, ctab_h, dum_h, pos_h, neg_h,
          tidx, cidx, nidx, trow0, trow1, crow0, crow1, nrow0, nrow1,
          posb, negb, sem0, sem1):
        wid = lax.axis_index("s") * 2 + lax.axis_index("c")
        pltpu.sync_copy(tid_h.at[pl.ds(wid * _BPW, _BPW)],
                        tidx.at[pl.ds(0, _BPW)])
        pltpu.sync_copy(cid_h.at[pl.ds(wid * _BPW, _BPW)],
                        cidx.at[pl.ds(0, _BPW)])
        pltpu.sync_copy(nid_h.at[pl.ds(wid * _BPW * _NEG, _BPW * _NEG)],
                        nidx.at[pl.ds(0, _BPW * _NEG)])

        lanes = lax.iota(jnp.int32, 16)
        bufs = ((trow0, crow0, nrow0, sem0), (trow1, crow1, nrow1, sem1))

        def fire(c, buf):
            trow, crow, nrow, sem = buf
            tvec = tidx[pl.ds(c * _CB, 16)]
            cvec = cidx[pl.ds(c * _CB, 16)]
            for i in range(_CB):
                traw = tvec[i]
                craw = cvec[i]
                pltpu.async_copy(
                    ttab_h.at[lax.shift_right_logical(traw, 3), traw & 7],
                    trow.at[i], sem)
                pltpu.async_copy(
                    ctab_h.at[lax.shift_right_logical(craw, 3), craw & 7],
                    crow.at[i], sem)
                nbase = c * _NEGC + i * _NEG
                nv0 = nidx[pl.ds(nbase, 16)]
                nv1 = nidx[pl.ds(nbase + 16, 16)]
                for n in range(_NEG):
                    nraw = nv0[n] if n < 16 else nv1[n - 16]
                    pltpu.async_copy(
                        ctab_h.at[lax.shift_right_logical(nraw, 3), nraw & 7],
                        nrow.at[i * _NEG + n], sem)

        def drain(buf):
            trow, crow, nrow, sem = buf
            pltpu.make_async_copy(dum_h.at[pl.ds(0, _CB)], trow, sem).wait()
            pltpu.make_async_copy(dum_h.at[pl.ds(0, _CB)], crow, sem).wait()
            pltpu.make_async_copy(dum_h, nrow, sem).wait()

        def compute(c, buf):
            trow, crow, nrow, _ = buf

            def lane_sum(vecs):
                return jnp.sum(vecs[0] + vecs[1] + vecs[2] + vecs[3])

            for g in range(_CB // 16):
                def item(i0, accs):
                    i = i0 + g * 16
                    onlane = lanes == i0
                    t = [trow[i, pl.ds(kk * 16, 16)] for kk in range(4)]
                    cv = [crow[i, pl.ds(kk * 16, 16)] for kk in range(4)]
                    new = [jnp.where(
                        onlane, lane_sum([t[kk] * cv[kk] for kk in range(4)]),
                        accs[0])]
                    for n in range(_NEG):
                        nv = [nrow[i * _NEG + n, pl.ds(kk * 16, 16)]
                              for kk in range(4)]
                        new.append(jnp.where(
                            onlane,
                            lane_sum([t[kk] * nv[kk] for kk in range(4)]),
                            accs[n + 1]))
                    return tuple(new)

                zeros = jnp.zeros((16,), jnp.float32)
                accs = lax.fori_loop(0, 16, item, (zeros,) * (_NEG + 1))
                posb[pl.ds(g * 16, 16)] = accs[0]
                for n in range(_NEG):
                    negb[pl.ds(n * _CB + g * 16, 16)] = accs[n + 1]

            base = wid * _BPW + c * _CB
            pltpu.sync_copy(posb, pos_h.at[pl.ds(base, _CB)])
            pltpu.sync_copy(negb, neg_h.at[pl.ds(base * _NEG, _NEGC)])

        fire(0, bufs[0])

        def pair(p, carry):
            for b in range(2):
                cur = p * 2 + b

                @pl.when(cur + 1 < _NCH)
                def _():
                    fire(cur + 1, bufs[1 - b])

                drain(bufs[b])
                compute(cur, bufs[b])
            return carry

        lax.fori_loop(0, _NCH // 2, pair, 0)

    return k(tid1, cid1, nid1, ttab, ctab, dum)


def _tc_loss(pos2d, neg2d):
    def body(p_ref, n_ref, o_ref):
        p = p_ref[...]
        n = n_ref[...]

        def logsig(x):
            return jnp.minimum(x, 0.0) - jnp.log1p(jnp.exp(-jnp.abs(x)))

        total = jnp.sum(logsig(p)) + jnp.sum(logsig(-n))
        o_ref[0, 0] = -total / _B

    out = pl.pallas_call(
        body,
        out_shape=jax.ShapeDtypeStruct((1, 1), jnp.float32),
        out_specs=pl.BlockSpec(memory_space=pltpu.SMEM),
    )(pos2d, neg2d)
    return out[0, 0]


def kernel(target_ids, context_ids, negative_ids, target_embed, context_embed):
    tid1 = target_ids.astype(jnp.int32)
    cid1 = context_ids.astype(jnp.int32)
    nid1 = negative_ids.astype(jnp.int32).reshape(-1)
    dum = jnp.zeros((_NEGC, _D), jnp.float32)
    pos, negf = _sc_scores(tid1, cid1, nid1,
                           target_embed.reshape(_V // 8, 8, _D),
                           context_embed.reshape(_V // 8, 8, _D), dum)
    return _tc_loss(pos.reshape(128, 128), negf.reshape(1280, 256))
